# baseline (speedup 1.0000x reference)
import functools

import jax
import jax.numpy as jnp
from jax import lax
from jax.experimental import pallas as pl
from jax.experimental.pallas import tpu as pltpu
from jax.experimental.pallas import tpu_sc as plsc

_K = 8
_C = 16
_QW = 16
_NG = 128
_GS = 32
_BIG = 1 << 30
_CN = 512


def _tc_body(x_ref, t_ref, o_ref, g_ref):
    x = x_ref[...]
    bq = x.shape[0]
    n = o_ref.shape[1]
    nchunks = n // _CN
    t = t_ref[...]
    ones_row = jnp.ones((1, x.shape[1]), jnp.float32)
    tn2 = lax.dot_general(
        ones_row, t * t, (((1,), (1,)), ((), ())),
        precision=lax.Precision.HIGHEST,
        preferred_element_type=jnp.float32)
    xh = x.astype(jnp.bfloat16)
    xl = (x - xh.astype(jnp.float32)).astype(jnp.bfloat16)
    th = t.astype(jnp.bfloat16)
    tl = (t - th.astype(jnp.float32)).astype(jnp.bfloat16)
    gacc = jnp.full((bq, _NG), jnp.inf, jnp.float32)
    dn = (((1,), (1,)), ((), ()))
    for c in range(nchunks):
        sl = slice(c * _CN, (c + 1) * _CN)
        xt = (lax.dot_general(xh, th[sl, :], dn,
                              preferred_element_type=jnp.float32)
              + lax.dot_general(xh, tl[sl, :], dn,
                                preferred_element_type=jnp.float32)
              + lax.dot_general(xl, th[sl, :], dn,
                                preferred_element_type=jnp.float32))
        d2c = tn2[:, sl] - 2.0 * xt
        o_ref[:, pl.ds(c * _CN, _CN)] = d2c
        for s in range(_CN // _NG):
            gacc = jnp.minimum(gacc, d2c[:, s * _NG:(s + 1) * _NG])
    g_ref[...] = gacc


def _dist_and_pyramid(inputs, X_train):
    q, d = inputs.shape
    n = X_train.shape[0]
    bq = 512
    return pl.pallas_call(
        _tc_body,
        grid=(q // bq,),
        in_specs=[
            pl.BlockSpec((bq, d), lambda i: (i, 0)),
            pl.BlockSpec((n, d), lambda i: (0, 0)),
        ],
        out_specs=[
            pl.BlockSpec((bq, n), lambda i: (i, 0)),
            pl.BlockSpec((bq, _NG), lambda i: (i, 0)),
        ],
        out_shape=[
            jax.ShapeDtypeStruct((q, n), jnp.float32),
            jax.ShapeDtypeStruct((q, _NG), jnp.float32),
        ],
    )(inputs, X_train)


def _sc_body(d2_hbm, g_hbm, y2_hbm, out_hbm, d_ref, gs_ref, gt_ref, idx_ref,
             rows_ref, obuf_ref, sem, dsem):
    w = lax.axis_index("s") * 2 + lax.axis_index("c")
    qbase = w * _QW
    lane = lax.broadcasted_iota(jnp.int32, (_QW,), 0)
    inf16 = jnp.full((_QW,), jnp.inf, jnp.float32)

    d2_cp = pltpu.async_copy(d2_hbm.at[pl.ds(qbase, _QW), :], d_ref, dsem)
    pltpu.sync_copy(g_hbm.at[pl.ds(qbase, _QW), :], gs_ref)

    def tr_body(r, col_v):
        gt_ref[r, :] = plsc.load_gather(gs_ref, [lane, col_v])
        return col_v + 1
    lax.fori_loop(0, _NG, tr_body, jnp.zeros((_QW,), jnp.int32), unroll=8)

    d2_cp.wait()

    offs = []
    ycps = []
    for k in range(_K):
        def argmin_body(r, carry):
            m, r_found = carry
            v = gt_ref[r, :]
            better = v < m
            return jnp.minimum(m, v), jnp.where(better, r, r_found)
        m, r_found = lax.fori_loop(
            0, _NG, argmin_body,
            (inf16, jnp.zeros((_QW,), jnp.int32)), unroll=8)

        def scan_body(c, carry):
            col_found, gmin = carry
            col = r_found + c * _NG
            val = plsc.load_gather(d_ref, [lane, col])
            is_t = (val == m) & (col_found == _BIG)
            col_found = jnp.where(is_t, col, col_found)
            gmin = jnp.minimum(gmin, jnp.where(is_t, jnp.inf, val))
            return col_found, gmin
        col_found, gmin = lax.fori_loop(
            0, _GS, scan_body,
            (jnp.full((_QW,), _BIG, jnp.int32), inf16), unroll=8)

        plsc.store_scatter(d_ref, [lane, col_found], inf16)
        plsc.store_scatter(gt_ref, [r_found, lane], gmin)
        plsc.store_scatter(idx_ref, [jnp.int32(k * _QW) + lane],
                           col_found >> 3)
        ycps.append(pltpu.async_copy(
            y2_hbm.at[idx_ref.at[pl.ds(k * _QW, _QW)]],
            rows_ref.at[pl.ds(k * _QW, _QW), :], sem))
        offs.append((col_found & 7) << 4)

    for q in range(_QW):
        for b in range(_C):
            obuf_ref[q, pl.ds(b * _C, _C)] = jnp.zeros((_C,), jnp.float32)
    accs = [jnp.zeros((_QW,), jnp.float32) for _ in range(_C)]
    for k in range(_K):
        ycps[k].wait()
        rowv = jnp.full((_QW,), k * _QW, jnp.int32) + lane
        for c in range(_C):
            accs[c] = accs[c] + plsc.load_gather(rows_ref,
                                                 [rowv, offs[k] + c])
    for c in range(_C):
        p = accs[c] * (1.0 / _K)
        cv = jnp.full((_QW,), c * _C, jnp.int32)
        plsc.store_scatter(obuf_ref, [lane, cv], 1.0 - p)
        plsc.store_scatter(obuf_ref, [lane, cv + 1], p)

    pltpu.sync_copy(obuf_ref, out_hbm.at[pl.ds(qbase, _QW), :])


def _sc_topk_counts(d2, g, y_train):
    q, n = d2.shape
    mesh = plsc.VectorSubcoreMesh(core_axis_name="c", subcore_axis_name="s")
    f = functools.partial(
        pl.kernel,
        out_type=jax.ShapeDtypeStruct((q, _C * _C), jnp.float32),
        mesh=mesh,
        scratch_types=[
            pltpu.VMEM((_QW, n), jnp.float32),
            pltpu.VMEM((_QW, _NG), jnp.float32),
            pltpu.VMEM((_NG, _QW), jnp.float32),
            pltpu.VMEM((_K * _QW,), jnp.int32),
            pltpu.VMEM((_K * _QW, 128), jnp.float32),
            pltpu.VMEM((_QW, _C * _C), jnp.float32),
            pltpu.SemaphoreType.DMA,
            pltpu.SemaphoreType.DMA,
        ],
        compiler_params=pltpu.CompilerParams(use_tc_tiling_on_sc=True,
                                             needs_layout_passes=False),
    )(_sc_body)
    n_tr, c_tr = y_train.shape
    y2 = y_train.reshape(n_tr * c_tr // 128, 128)
    return f(d2, g, y2)


def kernel(inputs, X_train, y_train):
    q = inputs.shape[0]
    d2, g = _dist_and_pyramid(inputs, X_train)
    out2d = _sc_topk_counts(d2, g, y_train)
    return out2d.reshape(q, _C, _C)
